# baseline (speedup 1.0000x reference)
import jax
import jax.numpy as jnp
from jax import lax
from jax.experimental import pallas as pl
from jax.experimental.pallas import tpu as pltpu
from jax.experimental.pallas import tpu_sc as plsc

N_NODES = 100000
N_EDGES = 6400000

NW = 32
CH = 1024
NCH = N_EDGES // CH
NP = 100352
SLC = NP // 16
R2 = NP // NW

_f32 = jnp.float32
_i32 = jnp.int32


def _rsqrt(t):
    bits = lax.bitcast_convert_type(t, _i32)
    y = lax.bitcast_convert_type(jnp.int32(0x5F3759DF) - (bits >> 1), _f32)
    for _ in range(3):
        y = y * (1.5 - 0.5 * t * y * y)
    return y


def _edge_pass(es1, ed1, posf, consts, zeros, fpart, epart,
               cat0, c3g0, c3s0, b30, g30,
               cat1, c3g1, c3s1, b31, g31,
               cbuf, facc, posi,
               sg0, sg1, ss0, ss1, sj0, sj1):
    c = lax.axis_index("c")
    s = lax.axis_index("s")
    wid = s * 2 + c

    set0 = (cat0, c3g0, c3s0, b30, g30, sg0, ss0, sj0)
    set1 = (cat1, c3g1, c3s1, b31, g31, sg1, ss1, sj1)

    sl3 = pl.ds(s * 3 * SLC, 3 * SLC)
    pltpu.sync_copy(zeros.at[pl.ds(0, 3 * SLC)], facc.at[sl3])
    pltpu.sync_copy(posf.at[sl3], posi.at[sl3])
    pltpu.sync_copy(consts, cbuf.at[pl.ds(0, 32)])
    cbuf[pl.ds(32, 16)] = jnp.zeros((16,), _f32)
    plsc.subcore_barrier()

    eps4 = cbuf[pl.ds(0, 16)]
    sig = cbuf[pl.ds(16, 16)]

    nk = (NCH - wid + NW - 1) // NW

    def gather_desc(S):
        return pltpu.make_async_copy(posi.at[S[1]], S[3], S[5])

    def scatter_drain(S):
        pltpu.make_async_copy(S[4], facc.at[S[2]], S[6]).wait()

    def scatter_issue(S):
        pltpu.async_copy(S[4], facc.at[S[2]], S[6], add=True)

    def idx_issue(S, j):
        cat = S[0]
        pltpu.async_copy(es1.at[pl.ds(j * CH, CH)], cat.at[pl.ds(0, CH)], S[7])
        pltpu.async_copy(ed1.at[pl.ds(j * CH, CH)], cat.at[pl.ds(CH, CH)], S[7])

    def idx_drain(S):
        cat = S[0]
        pltpu.make_async_copy(es1.at[pl.ds(0, CH)], cat.at[pl.ds(0, CH)], S[7]).wait()
        pltpu.make_async_copy(ed1.at[pl.ds(0, CH)], cat.at[pl.ds(CH, CH)], S[7]).wait()

    def build_c3g(S):
        cat, c3g = S[0], S[1]

        def bld(b, carry):
            cs = pl.ds(b * 16, 16)
            v3 = cat[cs] * 3
            c3g[cs] = v3
            c3g[pl.ds(2 * CH + b * 16, 16)] = v3 + 1
            c3g[pl.ds(4 * CH + b * 16, 16)] = v3 + 2
            return carry

        lax.fori_loop(0, 2 * CH // 16, bld, jnp.int32(0))

    def compute(S):
        cat, _, c3s, b3, g3 = S[0], S[1], S[2], S[3], S[4]

        def blk(b, acc):
            cs = pl.ds(b * 16, 16)
            cd = pl.ds(CH + b * 16, 16)
            csy = pl.ds(2 * CH + b * 16, 16)
            cdy = pl.ds(3 * CH + b * 16, 16)
            csz = pl.ds(4 * CH + b * 16, 16)
            cdz = pl.ds(5 * CH + b * 16, 16)
            vs3 = cat[cs] * 3
            c3s[cs] = vs3
            c3s[csy] = vs3 + 1
            c3s[csz] = vs3 + 2
            vd3 = cat[cd] * 3
            c3s[cd] = vd3
            c3s[cdy] = vd3 + 1
            c3s[cdz] = vd3 + 2
            dx = b3[cs] - b3[cd]
            dy = b3[csy] - b3[cdy]
            dz = b3[csz] - b3[cdz]
            t = dx * dx + dy * dy + dz * dz + 1e-12
            rin = _rsqrt(t)
            rr = t * rin
            qi = 1.0 / (rr + 1.0)
            inv = sig * qi
            i2 = inv * inv
            i6 = i2 * i2 * i2
            i12 = i6 * i6
            acc = acc + eps4 * (i12 - i6)
            dvdr = eps4 * (6.0 * i6 - 12.0 * i12) * qi
            cf = -0.5 * dvdr * rin
            hx = cf * dx
            hy = cf * dy
            hz = cf * dz
            g3[cs] = hx
            g3[cd] = -hx
            g3[csy] = hy
            g3[cdy] = -hy
            g3[csz] = hz
            g3[cdz] = -hz
            return acc

        eacc = lax.fori_loop(0, CH // 16, blk, cbuf[pl.ds(32, 16)])
        cbuf[pl.ds(32, 16)] = eacc

    def chunk_ops(k, S, T):
        @pl.when(k + 1 < nk)
        def _():
            idx_issue(T, wid + (k + 1) * NW)

        gather_desc(S).wait()

        @pl.when(k + 1 < nk)
        def _():
            idx_drain(T)
            build_c3g(T)
            gather_desc(T).start()

        compute(S)

        @pl.when(k >= 1)
        def _():
            scatter_drain(T)

        scatter_issue(S)

    pltpu.sync_copy(es1.at[pl.ds(wid * CH, CH)], cat0.at[pl.ds(0, CH)])
    pltpu.sync_copy(ed1.at[pl.ds(wid * CH, CH)], cat0.at[pl.ds(CH, CH)])
    build_c3g(set0)
    gather_desc(set0).start()

    def body(k, carry):
        @pl.when((k & 1) == 0)
        def _():
            chunk_ops(k, set0, set1)

        @pl.when((k & 1) == 1)
        def _():
            chunk_ops(k, set1, set0)

        return carry

    lax.fori_loop(0, nk, body, jnp.int32(0))

    last = (nk - 1) & 1

    @pl.when(last == 0)
    def _():
        scatter_drain(set0)

    @pl.when(last == 1)
    def _():
        scatter_drain(set1)

    pltpu.sync_copy(cbuf.at[pl.ds(32, 16)], epart.at[pl.ds(wid * 16, 16)])

    plsc.subcore_barrier()
    pltpu.sync_copy(facc.at[sl3], fpart.at[pl.ds(c * 3 * NP + s * 3 * SLC, 3 * SLC)])


def _tc_combine(a_ref, b_ref, e_ref, o_ref, eo_ref):
    o_ref[...] = a_ref[...] + b_ref[...]
    eo_ref[...] = jnp.sum(e_ref[...]).reshape(1, 1)


def kernel(pos, edge_index, epsilon, sigma):
    pos = pos.astype(_f32)
    posf = jnp.pad(pos.reshape(-1), (0, 3 * (NP - N_NODES)))
    es1 = edge_index[0]
    ed1 = edge_index[1]
    consts = jnp.concatenate([jnp.full((16,), 4.0 * epsilon, _f32),
                              jnp.full((16,), sigma, _f32)])
    zeros = jnp.zeros((3 * SLC,), _f32)

    mesh = plsc.VectorSubcoreMesh(core_axis_name="c", subcore_axis_name="s")

    fpart, epart = pl.kernel(
        _edge_pass,
        out_type=[
            jax.ShapeDtypeStruct((2 * 3 * NP,), _f32),
            jax.ShapeDtypeStruct((NW * 16,), _f32),
        ],
        mesh=mesh,
        scratch_types=(
            [
                pltpu.VMEM((2 * CH,), _i32),
                pltpu.VMEM((6 * CH,), _i32),
                pltpu.VMEM((6 * CH,), _i32),
                pltpu.VMEM((6 * CH,), _f32),
                pltpu.VMEM((6 * CH,), _f32),
                pltpu.VMEM((2 * CH,), _i32),
                pltpu.VMEM((6 * CH,), _i32),
                pltpu.VMEM((6 * CH,), _i32),
                pltpu.VMEM((6 * CH,), _f32),
                pltpu.VMEM((6 * CH,), _f32),
                pltpu.VMEM((48,), _f32),
                pltpu.VMEM_SHARED((3 * NP,), _f32),
                pltpu.VMEM_SHARED((3 * NP,), _f32),
            ]
            + [pltpu.SemaphoreType.DMA] * 6
        ),
    )(es1, ed1, posf, consts, zeros)

    rows = 3 * NP // 128
    a2 = fpart[:3 * NP].reshape(rows, 128)
    b2 = fpart[3 * NP:].reshape(rows, 128)
    e2 = epart.reshape(4, 128)
    force, etot = pl.pallas_call(
        _tc_combine,
        out_shape=[
            jax.ShapeDtypeStruct((rows, 128), _f32),
            jax.ShapeDtypeStruct((1, 1), _f32),
        ],
    )(a2, b2, e2)

    force = force.reshape(3 * NP)[:N_NODES * 3].reshape(N_NODES, 3)
    return etot[0, 0], force


# speedup vs baseline: 1.0689x; 1.0689x over previous
import jax
import jax.numpy as jnp
from jax import lax
from jax.experimental import pallas as pl
from jax.experimental.pallas import tpu as pltpu
from jax.experimental.pallas import tpu_sc as plsc

N_NODES = 100000
N_EDGES = 6400000

NW = 32
CH = 2048
NCH = N_EDGES // CH
NP = 100352
SLC = NP // 16
R2 = NP // NW

_f32 = jnp.float32
_i32 = jnp.int32


def _rsqrt(t):
    bits = lax.bitcast_convert_type(t, _i32)
    y = lax.bitcast_convert_type(jnp.int32(0x5F3759DF) - (bits >> 1), _f32)
    for _ in range(3):
        y = y * (1.5 - 0.5 * t * y * y)
    return y


def _edge_pass(es1, ed1, px, py, pz, consts, zeros, fpart, epart,
               cat0, cx0, cy0, cz0, bx0, by0, bz0, gx0, gy0, gz0,
               cat1, cx1, cy1, cz1, bx1, by1, bz1, gx1, gy1, gz1,
               cbuf, facc, spx, spy, spz,
               sg0, sg1, ss0, ss1, sj0, sj1):
    c = lax.axis_index("c")
    s = lax.axis_index("s")
    wid = s * 2 + c

    set0 = (cat0, cx0, cy0, cz0, bx0, by0, bz0, gx0, gy0, gz0, sg0, ss0, sj0)
    set1 = (cat1, cx1, cy1, cz1, bx1, by1, bz1, gx1, gy1, gz1, sg1, ss1, sj1)

    sl = pl.ds(s * SLC, SLC)
    pltpu.sync_copy(zeros.at[pl.ds(0, 3 * SLC)],
                    facc.at[pl.ds(s * 3 * SLC, 3 * SLC)])
    pltpu.sync_copy(px.at[sl], spx.at[sl])
    pltpu.sync_copy(py.at[sl], spy.at[sl])
    pltpu.sync_copy(pz.at[sl], spz.at[sl])
    pltpu.sync_copy(consts, cbuf.at[pl.ds(0, 32)])
    cbuf[pl.ds(32, 16)] = jnp.zeros((16,), _f32)
    plsc.subcore_barrier()

    eps4 = cbuf[pl.ds(0, 16)]
    sig = cbuf[pl.ds(16, 16)]

    nk = (NCH - wid + NW - 1) // NW

    def gather_descs(S):
        cat, bx, by, bz = S[0], S[4], S[5], S[6]
        return [pltpu.make_async_copy(spx.at[cat], bx, S[10]),
                pltpu.make_async_copy(spy.at[cat], by, S[10]),
                pltpu.make_async_copy(spz.at[cat], bz, S[10])]

    def scatter_drain(S):
        cx, cy, cz, gx, gy, gz = S[1], S[2], S[3], S[7], S[8], S[9]
        pltpu.make_async_copy(gx, facc.at[cx], S[11]).wait()
        pltpu.make_async_copy(gy, facc.at[cy], S[11]).wait()
        pltpu.make_async_copy(gz, facc.at[cz], S[11]).wait()

    def scatter_issue(S):
        cx, cy, cz, gx, gy, gz = S[1], S[2], S[3], S[7], S[8], S[9]
        pltpu.async_copy(gx, facc.at[cx], S[11], add=True)
        pltpu.async_copy(gy, facc.at[cy], S[11], add=True)
        pltpu.async_copy(gz, facc.at[cz], S[11], add=True)

    def idx_issue(S, j):
        cat = S[0]
        pltpu.async_copy(es1.at[pl.ds(j * CH, CH)], cat.at[pl.ds(0, CH)], S[12])
        pltpu.async_copy(ed1.at[pl.ds(j * CH, CH)], cat.at[pl.ds(CH, CH)], S[12])

    def idx_drain(S):
        cat = S[0]
        pltpu.make_async_copy(es1.at[pl.ds(0, CH)], cat.at[pl.ds(0, CH)], S[12]).wait()
        pltpu.make_async_copy(ed1.at[pl.ds(0, CH)], cat.at[pl.ds(CH, CH)], S[12]).wait()

    def compute(S):
        cat, cx, cy, cz = S[0], S[1], S[2], S[3]
        bx, by, bz, gx, gy, gz = S[4], S[5], S[6], S[7], S[8], S[9]

        def blk(b, acc):
            cs = pl.ds(b * 16, 16)
            cd = pl.ds(CH + b * 16, 16)
            vs3 = cat[cs] * 3
            cx[cs] = vs3
            cy[cs] = vs3 + 1
            cz[cs] = vs3 + 2
            vd3 = cat[cd] * 3
            cx[cd] = vd3
            cy[cd] = vd3 + 1
            cz[cd] = vd3 + 2
            dx = bx[cs] - bx[cd]
            dy = by[cs] - by[cd]
            dz = bz[cs] - bz[cd]
            t = dx * dx + dy * dy + dz * dz + 1e-12
            rin = _rsqrt(t)
            rr = t * rin
            qi = 1.0 / (rr + 1.0)
            inv = sig * qi
            i2 = inv * inv
            i6 = i2 * i2 * i2
            i12 = i6 * i6
            acc = acc + eps4 * (i12 - i6)
            dvdr = eps4 * (6.0 * i6 - 12.0 * i12) * qi
            cf = -0.5 * dvdr * rin
            hx = cf * dx
            hy = cf * dy
            hz = cf * dz
            gx[cs] = hx
            gy[cs] = hy
            gz[cs] = hz
            gx[cd] = -hx
            gy[cd] = -hy
            gz[cd] = -hz
            return acc

        eacc = lax.fori_loop(0, CH // 16, blk, cbuf[pl.ds(32, 16)])
        cbuf[pl.ds(32, 16)] = eacc

    def chunk_ops(k, S, T):
        @pl.when(k >= 1)
        def _():
            scatter_drain(T)

        @pl.when(k + 1 < nk)
        def _():
            idx_issue(T, wid + (k + 1) * NW)

        for d in gather_descs(S):
            d.wait()

        @pl.when(k + 1 < nk)
        def _():
            idx_drain(T)
            for d in gather_descs(T):
                d.start()

        compute(S)
        scatter_issue(S)

    pltpu.sync_copy(es1.at[pl.ds(wid * CH, CH)], cat0.at[pl.ds(0, CH)])
    pltpu.sync_copy(ed1.at[pl.ds(wid * CH, CH)], cat0.at[pl.ds(CH, CH)])
    for d in gather_descs(set0):
        d.start()

    def body(k, carry):
        @pl.when((k & 1) == 0)
        def _():
            chunk_ops(k, set0, set1)

        @pl.when((k & 1) == 1)
        def _():
            chunk_ops(k, set1, set0)

        return carry

    lax.fori_loop(0, nk, body, jnp.int32(0))

    last = (nk - 1) & 1

    @pl.when(last == 0)
    def _():
        scatter_drain(set0)

    @pl.when(last == 1)
    def _():
        scatter_drain(set1)

    pltpu.sync_copy(cbuf.at[pl.ds(32, 16)], epart.at[pl.ds(wid * 16, 16)])

    plsc.subcore_barrier()
    pltpu.sync_copy(facc.at[pl.ds(s * 3 * SLC, 3 * SLC)],
                    fpart.at[pl.ds(c * 3 * NP + s * 3 * SLC, 3 * SLC)])


def _tc_combine(a_ref, b_ref, e_ref, o_ref, eo_ref):
    o_ref[...] = a_ref[...] + b_ref[...]
    eo_ref[...] = jnp.sum(e_ref[...]).reshape(1, 1)


def kernel(pos, edge_index, epsilon, sigma):
    pos = pos.astype(_f32)
    px = jnp.pad(pos[:, 0], (0, NP - N_NODES))
    py = jnp.pad(pos[:, 1], (0, NP - N_NODES))
    pz = jnp.pad(pos[:, 2], (0, NP - N_NODES))
    es1 = edge_index[0]
    ed1 = edge_index[1]
    consts = jnp.concatenate([jnp.full((16,), 4.0 * epsilon, _f32),
                              jnp.full((16,), sigma, _f32)])
    zeros = jnp.zeros((3 * SLC,), _f32)

    mesh = plsc.VectorSubcoreMesh(core_axis_name="c", subcore_axis_name="s")

    fpart, epart = pl.kernel(
        _edge_pass,
        out_type=[
            jax.ShapeDtypeStruct((2 * 3 * NP,), _f32),
            jax.ShapeDtypeStruct((NW * 16,), _f32),
        ],
        mesh=mesh,
        scratch_types=(
            [pltpu.VMEM((2 * CH,), _i32)] * 4
            + [pltpu.VMEM((2 * CH,), _f32)] * 6
            + [pltpu.VMEM((2 * CH,), _i32)] * 4
            + [pltpu.VMEM((2 * CH,), _f32)] * 6
            + [
                pltpu.VMEM((48,), _f32),
                pltpu.VMEM_SHARED((3 * NP,), _f32),
                pltpu.VMEM_SHARED((NP,), _f32),
                pltpu.VMEM_SHARED((NP,), _f32),
                pltpu.VMEM_SHARED((NP,), _f32),
            ]
            + [pltpu.SemaphoreType.DMA] * 6
        ),
    )(es1, ed1, px, py, pz, consts, zeros)

    rows = 3 * NP // 128
    a2 = fpart[:3 * NP].reshape(rows, 128)
    b2 = fpart[3 * NP:].reshape(rows, 128)
    e2 = epart.reshape(4, 128)
    force, etot = pl.pallas_call(
        _tc_combine,
        out_shape=[
            jax.ShapeDtypeStruct((rows, 128), _f32),
            jax.ShapeDtypeStruct((1, 1), _f32),
        ],
    )(a2, b2, e2)

    force = force.reshape(3 * NP)[:N_NODES * 3].reshape(N_NODES, 3)
    return etot[0, 0], force


# speedup vs baseline: 1.1660x; 1.0908x over previous
import jax
import jax.numpy as jnp
from jax import lax
from jax.experimental import pallas as pl
from jax.experimental.pallas import tpu as pltpu
from jax.experimental.pallas import tpu_sc as plsc

N_NODES = 100000
N_EDGES = 6400000

NW = 32
CH = 2048
NCH = N_EDGES // CH
NP = 100352
SLC = NP // 16
R2 = NP // NW

_f32 = jnp.float32
_i32 = jnp.int32


def _rsqrt(t):
    bits = lax.bitcast_convert_type(t, _i32)
    y = lax.bitcast_convert_type(jnp.int32(0x5F3759DF) - (bits >> 1), _f32)
    for _ in range(3):
        y = y * (1.5 - 0.5 * t * y * y)
    return y


def _edge_pass(es1, ed1, px, py, pz, consts, zeros, fpart, epart,
               cat0, cx0, cy0, cz0, bx0, by0, bz0, gx0, gy0, gz0,
               cat1, cx1, cy1, cz1, bx1, by1, bz1, gx1, gy1, gz1,
               cbuf, facc, spx, spy, spz,
               sg0, sg1, ss0, ss1, sj0, sj1):
    c = lax.axis_index("c")
    s = lax.axis_index("s")
    wid = s * 2 + c

    set0 = (cat0, cx0, cy0, cz0, bx0, by0, bz0, gx0, gy0, gz0, sg0, ss0, sj0)
    set1 = (cat1, cx1, cy1, cz1, bx1, by1, bz1, gx1, gy1, gz1, sg1, ss1, sj1)

    sl = pl.ds(s * SLC, SLC)
    pltpu.sync_copy(zeros.at[pl.ds(0, 3 * SLC)],
                    facc.at[pl.ds(s * 3 * SLC, 3 * SLC)])
    pltpu.sync_copy(px.at[sl], spx.at[sl])
    pltpu.sync_copy(py.at[sl], spy.at[sl])
    pltpu.sync_copy(pz.at[sl], spz.at[sl])
    pltpu.sync_copy(consts, cbuf.at[pl.ds(0, 32)])
    cbuf[pl.ds(32, 16)] = jnp.zeros((16,), _f32)
    plsc.subcore_barrier()

    eps4 = cbuf[pl.ds(0, 16)]
    sig = cbuf[pl.ds(16, 16)]

    nk = (NCH - wid + NW - 1) // NW

    def gather_descs(S):
        cat, bx, by, bz = S[0], S[4], S[5], S[6]
        return [pltpu.make_async_copy(spx.at[cat], bx, S[10]),
                pltpu.make_async_copy(spy.at[cat], by, S[10]),
                pltpu.make_async_copy(spz.at[cat], bz, S[10])]

    def scatter_drain(S):
        cx, cy, cz, gx, gy, gz = S[1], S[2], S[3], S[7], S[8], S[9]
        pltpu.make_async_copy(gx, facc.at[cx], S[11]).wait()
        pltpu.make_async_copy(gy, facc.at[cy], S[11]).wait()
        pltpu.make_async_copy(gz, facc.at[cz], S[11]).wait()

    def scatter_issue(S):
        cx, cy, cz, gx, gy, gz = S[1], S[2], S[3], S[7], S[8], S[9]
        pltpu.async_copy(gx, facc.at[cx], S[11], add=True)
        pltpu.async_copy(gy, facc.at[cy], S[11], add=True)
        pltpu.async_copy(gz, facc.at[cz], S[11], add=True)

    def idx_issue(S, j):
        cat = S[0]
        pltpu.async_copy(es1.at[pl.ds(j * CH, CH)], cat.at[pl.ds(0, CH)], S[12])
        pltpu.async_copy(ed1.at[pl.ds(j * CH, CH)], cat.at[pl.ds(CH, CH)], S[12])

    def idx_drain(S):
        cat = S[0]
        pltpu.make_async_copy(es1.at[pl.ds(0, CH)], cat.at[pl.ds(0, CH)], S[12]).wait()
        pltpu.make_async_copy(ed1.at[pl.ds(0, CH)], cat.at[pl.ds(CH, CH)], S[12]).wait()

    def compute(S):
        cat, cx, cy, cz = S[0], S[1], S[2], S[3]
        bx, by, bz, gx, gy, gz = S[4], S[5], S[6], S[7], S[8], S[9]

        def blk(b, acc):
            cs = pl.ds(b * 16, 16)
            cd = pl.ds(CH + b * 16, 16)
            vs3 = cat[cs] * 3
            cx[cs] = vs3
            cy[cs] = vs3 + 1
            cz[cs] = vs3 + 2
            vd3 = cat[cd] * 3
            cx[cd] = vd3
            cy[cd] = vd3 + 1
            cz[cd] = vd3 + 2
            dx = bx[cs] - bx[cd]
            dy = by[cs] - by[cd]
            dz = bz[cs] - bz[cd]
            t = dx * dx + dy * dy + dz * dz + 1e-12
            rin = _rsqrt(t)
            rr = t * rin
            qi = 1.0 / (rr + 1.0)
            inv = sig * qi
            i2 = inv * inv
            i6 = i2 * i2 * i2
            i12 = i6 * i6
            acc = acc + eps4 * (i12 - i6)
            dvdr = eps4 * (6.0 * i6 - 12.0 * i12) * qi
            cf = -0.5 * dvdr * rin
            hx = cf * dx
            hy = cf * dy
            hz = cf * dz
            gx[cs] = hx
            gy[cs] = hy
            gz[cs] = hz
            gx[cd] = -hx
            gy[cd] = -hy
            gz[cd] = -hz
            return acc

        eacc = lax.fori_loop(0, CH // 16, blk, cbuf[pl.ds(32, 16)])
        cbuf[pl.ds(32, 16)] = eacc

    def chunk_ops(k, S, T):
        @pl.when(k + 1 < nk)
        def _():
            idx_issue(T, wid + (k + 1) * NW)

        for d in gather_descs(S):
            d.wait()

        @pl.when(k + 1 < nk)
        def _():
            idx_drain(T)
            for d in gather_descs(T):
                d.start()

        compute(S)

        @pl.when(k >= 1)
        def _():
            scatter_drain(T)

        scatter_issue(S)

    pltpu.sync_copy(es1.at[pl.ds(wid * CH, CH)], cat0.at[pl.ds(0, CH)])
    pltpu.sync_copy(ed1.at[pl.ds(wid * CH, CH)], cat0.at[pl.ds(CH, CH)])
    for d in gather_descs(set0):
        d.start()

    def body(k, carry):
        @pl.when((k & 1) == 0)
        def _():
            chunk_ops(k, set0, set1)

        @pl.when((k & 1) == 1)
        def _():
            chunk_ops(k, set1, set0)

        return carry

    lax.fori_loop(0, nk, body, jnp.int32(0))

    last = (nk - 1) & 1

    @pl.when(last == 0)
    def _():
        scatter_drain(set0)

    @pl.when(last == 1)
    def _():
        scatter_drain(set1)

    pltpu.sync_copy(cbuf.at[pl.ds(32, 16)], epart.at[pl.ds(wid * 16, 16)])

    plsc.subcore_barrier()
    pltpu.sync_copy(facc.at[pl.ds(s * 3 * SLC, 3 * SLC)],
                    fpart.at[pl.ds(c * 3 * NP + s * 3 * SLC, 3 * SLC)])


def _tc_combine(a_ref, b_ref, e_ref, o_ref, eo_ref):
    o_ref[...] = a_ref[...] + b_ref[...]
    eo_ref[...] = jnp.sum(e_ref[...]).reshape(1, 1)


def kernel(pos, edge_index, epsilon, sigma):
    pos = pos.astype(_f32)
    px = jnp.pad(pos[:, 0], (0, NP - N_NODES))
    py = jnp.pad(pos[:, 1], (0, NP - N_NODES))
    pz = jnp.pad(pos[:, 2], (0, NP - N_NODES))
    es1 = edge_index[0]
    ed1 = edge_index[1]
    consts = jnp.concatenate([jnp.full((16,), 4.0 * epsilon, _f32),
                              jnp.full((16,), sigma, _f32)])
    zeros = jnp.zeros((3 * SLC,), _f32)

    mesh = plsc.VectorSubcoreMesh(core_axis_name="c", subcore_axis_name="s")

    fpart, epart = pl.kernel(
        _edge_pass,
        out_type=[
            jax.ShapeDtypeStruct((2 * 3 * NP,), _f32),
            jax.ShapeDtypeStruct((NW * 16,), _f32),
        ],
        mesh=mesh,
        scratch_types=(
            [pltpu.VMEM((2 * CH,), _i32)] * 4
            + [pltpu.VMEM((2 * CH,), _f32)] * 6
            + [pltpu.VMEM((2 * CH,), _i32)] * 4
            + [pltpu.VMEM((2 * CH,), _f32)] * 6
            + [
                pltpu.VMEM((48,), _f32),
                pltpu.VMEM_SHARED((3 * NP,), _f32),
                pltpu.VMEM_SHARED((NP,), _f32),
                pltpu.VMEM_SHARED((NP,), _f32),
                pltpu.VMEM_SHARED((NP,), _f32),
            ]
            + [pltpu.SemaphoreType.DMA] * 6
        ),
    )(es1, ed1, px, py, pz, consts, zeros)

    rows = 3 * NP // 128
    a2 = fpart[:3 * NP].reshape(rows, 128)
    b2 = fpart[3 * NP:].reshape(rows, 128)
    e2 = epart.reshape(4, 128)
    force, etot = pl.pallas_call(
        _tc_combine,
        out_shape=[
            jax.ShapeDtypeStruct((rows, 128), _f32),
            jax.ShapeDtypeStruct((1, 1), _f32),
        ],
    )(a2, b2, e2)

    force = force.reshape(3 * NP)[:N_NODES * 3].reshape(N_NODES, 3)
    return etot[0, 0], force


# speedup vs baseline: 1.1960x; 1.0258x over previous
import jax
import jax.numpy as jnp
from jax import lax
from jax.experimental import pallas as pl
from jax.experimental.pallas import tpu as pltpu
from jax.experimental.pallas import tpu_sc as plsc

N_NODES = 100000
N_EDGES = 6400000

NW = 32
CH = 2048
NCH = N_EDGES // CH
NP = 100352
SLC = NP // 16
R2 = NP // NW

_f32 = jnp.float32
_i32 = jnp.int32


def _rsqrt(t):
    bits = lax.bitcast_convert_type(t, _i32)
    y = lax.bitcast_convert_type(jnp.int32(0x5F3759DF) - (bits >> 1), _f32)
    for _ in range(3):
        y = y * (1.5 - 0.5 * t * y * y)
    return y


def _edge_pass(eif, px, py, pz, consts, zeros, fpart, epart,
               cat0, cx0, cy0, cz0, bx0, by0, bz0, gx0, gy0, gz0,
               cat1, cx1, cy1, cz1, bx1, by1, bz1, gx1, gy1, gz1,
               cbuf, facc, spx, spy, spz,
               sg0, sg1, ss0, ss1, sj0, sj1):
    c = lax.axis_index("c")
    s = lax.axis_index("s")
    wid = s * 2 + c

    set0 = (cat0, cx0, cy0, cz0, bx0, by0, bz0, gx0, gy0, gz0, sg0, ss0, sj0)
    set1 = (cat1, cx1, cy1, cz1, bx1, by1, bz1, gx1, gy1, gz1, sg1, ss1, sj1)

    sl = pl.ds(s * SLC, SLC)
    pltpu.sync_copy(zeros.at[pl.ds(0, 3 * SLC)],
                    facc.at[pl.ds(s * 3 * SLC, 3 * SLC)])
    pltpu.sync_copy(px.at[sl], spx.at[sl])
    pltpu.sync_copy(py.at[sl], spy.at[sl])
    pltpu.sync_copy(pz.at[sl], spz.at[sl])
    pltpu.sync_copy(consts, cbuf.at[pl.ds(0, 32)])
    cbuf[pl.ds(32, 16)] = jnp.zeros((16,), _f32)
    plsc.subcore_barrier()

    eps4 = cbuf[pl.ds(0, 16)]
    sig = cbuf[pl.ds(16, 16)]

    nk = (NCH - wid + NW - 1) // NW

    def gather_descs(S):
        cat, bx, by, bz = S[0], S[4], S[5], S[6]
        return [pltpu.make_async_copy(spx.at[cat], bx, S[10]),
                pltpu.make_async_copy(spy.at[cat], by, S[10]),
                pltpu.make_async_copy(spz.at[cat], bz, S[10])]

    def scatter_drain(S):
        cx, cy, cz, gx, gy, gz = S[1], S[2], S[3], S[7], S[8], S[9]
        pltpu.make_async_copy(gx, facc.at[cx], S[11]).wait()
        pltpu.make_async_copy(gy, facc.at[cy], S[11]).wait()
        pltpu.make_async_copy(gz, facc.at[cz], S[11]).wait()

    def scatter_issue(S):
        cx, cy, cz, gx, gy, gz = S[1], S[2], S[3], S[7], S[8], S[9]
        pltpu.async_copy(gx, facc.at[cx], S[11], add=True)
        pltpu.async_copy(gy, facc.at[cy], S[11], add=True)
        pltpu.async_copy(gz, facc.at[cz], S[11], add=True)

    def idx_issue(S, j):
        cat = S[0]
        pltpu.async_copy(eif.at[pl.ds(j * CH, CH)], cat.at[pl.ds(0, CH)], S[12])
        pltpu.async_copy(eif.at[pl.ds(N_EDGES + j * CH, CH)],
                         cat.at[pl.ds(CH, CH)], S[12])

    def idx_drain(S):
        cat = S[0]
        pltpu.make_async_copy(eif.at[pl.ds(0, CH)], cat.at[pl.ds(0, CH)], S[12]).wait()
        pltpu.make_async_copy(eif.at[pl.ds(0, CH)], cat.at[pl.ds(CH, CH)], S[12]).wait()

    def compute(S):
        cat, cx, cy, cz = S[0], S[1], S[2], S[3]
        bx, by, bz, gx, gy, gz = S[4], S[5], S[6], S[7], S[8], S[9]

        def blk(b, acc):
            cs = pl.ds(b * 16, 16)
            cd = pl.ds(CH + b * 16, 16)
            vs3 = cat[cs] * 3
            cx[cs] = vs3
            cy[cs] = vs3 + 1
            cz[cs] = vs3 + 2
            vd3 = cat[cd] * 3
            cx[cd] = vd3
            cy[cd] = vd3 + 1
            cz[cd] = vd3 + 2
            dx = bx[cs] - bx[cd]
            dy = by[cs] - by[cd]
            dz = bz[cs] - bz[cd]
            t = dx * dx + dy * dy + dz * dz + 1e-12
            rin = _rsqrt(t)
            rr = t * rin
            qi = 1.0 / (rr + 1.0)
            inv = sig * qi
            i2 = inv * inv
            i6 = i2 * i2 * i2
            i12 = i6 * i6
            acc = acc + eps4 * (i12 - i6)
            dvdr = eps4 * (6.0 * i6 - 12.0 * i12) * qi
            cf = -0.5 * dvdr * rin
            hx = cf * dx
            hy = cf * dy
            hz = cf * dz
            gx[cs] = hx
            gy[cs] = hy
            gz[cs] = hz
            gx[cd] = -hx
            gy[cd] = -hy
            gz[cd] = -hz
            return acc

        eacc = lax.fori_loop(0, CH // 16, blk, cbuf[pl.ds(32, 16)])
        cbuf[pl.ds(32, 16)] = eacc

    def chunk_ops(k, S, T):
        @pl.when(k + 1 < nk)
        def _():
            idx_issue(T, wid + (k + 1) * NW)

        for d in gather_descs(S):
            d.wait()

        @pl.when(k + 1 < nk)
        def _():
            idx_drain(T)
            for d in gather_descs(T):
                d.start()

        compute(S)

        @pl.when(k >= 1)
        def _():
            scatter_drain(T)

        scatter_issue(S)

    pltpu.sync_copy(eif.at[pl.ds(wid * CH, CH)], cat0.at[pl.ds(0, CH)])
    pltpu.sync_copy(eif.at[pl.ds(N_EDGES + wid * CH, CH)],
                    cat0.at[pl.ds(CH, CH)])
    for d in gather_descs(set0):
        d.start()

    def body(k, carry):
        @pl.when((k & 1) == 0)
        def _():
            chunk_ops(k, set0, set1)

        @pl.when((k & 1) == 1)
        def _():
            chunk_ops(k, set1, set0)

        return carry

    lax.fori_loop(0, nk, body, jnp.int32(0))

    last = (nk - 1) & 1

    @pl.when(last == 0)
    def _():
        scatter_drain(set0)

    @pl.when(last == 1)
    def _():
        scatter_drain(set1)

    pltpu.sync_copy(cbuf.at[pl.ds(32, 16)], epart.at[pl.ds(wid * 16, 16)])

    plsc.subcore_barrier()
    pltpu.sync_copy(facc.at[pl.ds(s * 3 * SLC, 3 * SLC)],
                    fpart.at[pl.ds(c * 3 * NP + s * 3 * SLC, 3 * SLC)])


def _tc_combine(a_ref, b_ref, e_ref, o_ref, eo_ref):
    o_ref[...] = a_ref[...] + b_ref[...]
    eo_ref[...] = jnp.sum(e_ref[...]).reshape(1, 1)


def kernel(pos, edge_index, epsilon, sigma):
    pos = pos.astype(_f32)
    px = jnp.pad(pos[:, 0], (0, NP - N_NODES))
    py = jnp.pad(pos[:, 1], (0, NP - N_NODES))
    pz = jnp.pad(pos[:, 2], (0, NP - N_NODES))
    eif = edge_index.reshape(-1)
    consts = jnp.concatenate([jnp.full((16,), 4.0 * epsilon, _f32),
                              jnp.full((16,), sigma, _f32)])
    zeros = jnp.zeros((3 * SLC,), _f32)

    mesh = plsc.VectorSubcoreMesh(core_axis_name="c", subcore_axis_name="s")

    fpart, epart = pl.kernel(
        _edge_pass,
        out_type=[
            jax.ShapeDtypeStruct((2 * 3 * NP,), _f32),
            jax.ShapeDtypeStruct((NW * 16,), _f32),
        ],
        mesh=mesh,
        scratch_types=(
            [pltpu.VMEM((2 * CH,), _i32)] * 4
            + [pltpu.VMEM((2 * CH,), _f32)] * 6
            + [pltpu.VMEM((2 * CH,), _i32)] * 4
            + [pltpu.VMEM((2 * CH,), _f32)] * 6
            + [
                pltpu.VMEM((48,), _f32),
                pltpu.VMEM_SHARED((3 * NP,), _f32),
                pltpu.VMEM_SHARED((NP,), _f32),
                pltpu.VMEM_SHARED((NP,), _f32),
                pltpu.VMEM_SHARED((NP,), _f32),
            ]
            + [pltpu.SemaphoreType.DMA] * 6
        ),
    )(eif, px, py, pz, consts, zeros)

    rows = 3 * NP // 128
    a2 = fpart[:3 * NP].reshape(rows, 128)
    b2 = fpart[3 * NP:].reshape(rows, 128)
    e2 = epart.reshape(4, 128)
    force, etot = pl.pallas_call(
        _tc_combine,
        out_shape=[
            jax.ShapeDtypeStruct((rows, 128), _f32),
            jax.ShapeDtypeStruct((1, 1), _f32),
        ],
    )(a2, b2, e2)

    force = force.reshape(3 * NP)[:N_NODES * 3].reshape(N_NODES, 3)
    return etot[0, 0], force


# speedup vs baseline: 1.1974x; 1.0011x over previous
import jax
import jax.numpy as jnp
from jax import lax
from jax.experimental import pallas as pl
from jax.experimental.pallas import tpu as pltpu
from jax.experimental.pallas import tpu_sc as plsc

N_NODES = 100000
N_EDGES = 6400000

NW = 32
CH = 2048
NCH = N_EDGES // CH
NP = 100352
SLC = NP // 16
R2 = NP // NW

_f32 = jnp.float32
_i32 = jnp.int32


def _rsqrt(t):
    bits = lax.bitcast_convert_type(t, _i32)
    y = lax.bitcast_convert_type(jnp.int32(0x5F3759DF) - (bits >> 1), _f32)
    for _ in range(3):
        y = y * (1.5 - 0.5 * t * y * y)
    return y


def _edge_pass(eif, px, py, pz, consts, zeros, fpart, epart,
               cat0, cx0, cy0, cz0, bx0, by0, bz0, gx0, gy0, gz0,
               cat1, cx1, cy1, cz1, bx1, by1, bz1, gx1, gy1, gz1,
               cbuf, facc, spx, spy, spz,
               sg0, sg1, ss0, ss1, sj0, sj1):
    c = lax.axis_index("c")
    s = lax.axis_index("s")
    wid = s * 2 + c

    set0 = (cat0, cx0, cy0, cz0, bx0, by0, bz0, gx0, gy0, gz0, sg0, ss0, sj0)
    set1 = (cat1, cx1, cy1, cz1, bx1, by1, bz1, gx1, gy1, gz1, sg1, ss1, sj1)

    sl = pl.ds(s * SLC, SLC)
    pltpu.sync_copy(zeros.at[pl.ds(0, 3 * SLC)],
                    facc.at[pl.ds(s * 3 * SLC, 3 * SLC)])
    pltpu.sync_copy(px.at[sl], spx.at[sl])
    pltpu.sync_copy(py.at[sl], spy.at[sl])
    pltpu.sync_copy(pz.at[sl], spz.at[sl])
    pltpu.sync_copy(consts, cbuf.at[pl.ds(0, 32)])
    cbuf[pl.ds(32, 16)] = jnp.zeros((16,), _f32)
    plsc.subcore_barrier()

    eps4 = cbuf[pl.ds(0, 16)]
    sig = cbuf[pl.ds(16, 16)]

    nk = (NCH - wid + NW - 1) // NW

    def gather_descs(S):
        cat, bx, by, bz = S[0], S[4], S[5], S[6]
        return [pltpu.make_async_copy(spx.at[cat], bx, S[10]),
                pltpu.make_async_copy(spy.at[cat], by, S[10]),
                pltpu.make_async_copy(spz.at[cat], bz, S[10])]

    def scatter_drain(S):
        cx, cy, cz, gx, gy, gz = S[1], S[2], S[3], S[7], S[8], S[9]
        pltpu.make_async_copy(gx, facc.at[cx], S[11]).wait()
        pltpu.make_async_copy(gy, facc.at[cy], S[11]).wait()
        pltpu.make_async_copy(gz, facc.at[cz], S[11]).wait()

    def scatter_issue(S):
        cx, cy, cz, gx, gy, gz = S[1], S[2], S[3], S[7], S[8], S[9]
        pltpu.async_copy(gx, facc.at[cx], S[11], add=True)
        pltpu.async_copy(gy, facc.at[cy], S[11], add=True)
        pltpu.async_copy(gz, facc.at[cz], S[11], add=True)

    def idx_issue(S, j):
        cat = S[0]
        pltpu.async_copy(eif.at[pl.ds(j * CH, CH)], cat.at[pl.ds(0, CH)], S[12])
        pltpu.async_copy(eif.at[pl.ds(N_EDGES + j * CH, CH)],
                         cat.at[pl.ds(CH, CH)], S[12])

    def idx_drain(S):
        cat = S[0]
        pltpu.make_async_copy(eif.at[pl.ds(0, CH)], cat.at[pl.ds(0, CH)], S[12]).wait()
        pltpu.make_async_copy(eif.at[pl.ds(0, CH)], cat.at[pl.ds(CH, CH)], S[12]).wait()

    def compute(S):
        cat, cx, cy, cz = S[0], S[1], S[2], S[3]
        bx, by, bz, gx, gy, gz = S[4], S[5], S[6], S[7], S[8], S[9]

        def blk(b, acc):
            cs = pl.ds(b * 16, 16)
            cd = pl.ds(CH + b * 16, 16)
            vs3 = cat[cs] * 3
            cx[cs] = vs3
            cy[cs] = vs3 + 1
            cz[cs] = vs3 + 2
            vd3 = cat[cd] * 3
            cx[cd] = vd3
            cy[cd] = vd3 + 1
            cz[cd] = vd3 + 2
            dx = bx[cs] - bx[cd]
            dy = by[cs] - by[cd]
            dz = bz[cs] - bz[cd]
            t = dx * dx + dy * dy + dz * dz + 1e-12
            rin = _rsqrt(t)
            rr = t * rin
            qi = 1.0 / (rr + 1.0)
            inv = sig * qi
            i2 = inv * inv
            i6 = i2 * i2 * i2
            i12 = i6 * i6
            acc = acc + eps4 * (i12 - i6)
            dvdr = eps4 * (6.0 * i6 - 12.0 * i12) * qi
            cf = -0.5 * dvdr * rin
            hx = cf * dx
            hy = cf * dy
            hz = cf * dz
            gx[cs] = hx
            gy[cs] = hy
            gz[cs] = hz
            gx[cd] = -hx
            gy[cd] = -hy
            gz[cd] = -hz
            return acc

        eacc = lax.fori_loop(0, CH // 16, blk, cbuf[pl.ds(32, 16)])
        cbuf[pl.ds(32, 16)] = eacc

    def chunk_ops(k, S, T):
        @pl.when(k + 1 < nk)
        def _():
            idx_issue(T, wid + (k + 1) * NW)
            idx_drain(T)
            for d in gather_descs(T):
                d.start()

        for d in gather_descs(S):
            d.wait()

        compute(S)

        @pl.when(k >= 1)
        def _():
            scatter_drain(T)

        scatter_issue(S)

    pltpu.sync_copy(eif.at[pl.ds(wid * CH, CH)], cat0.at[pl.ds(0, CH)])
    pltpu.sync_copy(eif.at[pl.ds(N_EDGES + wid * CH, CH)],
                    cat0.at[pl.ds(CH, CH)])
    for d in gather_descs(set0):
        d.start()

    def body(k, carry):
        @pl.when((k & 1) == 0)
        def _():
            chunk_ops(k, set0, set1)

        @pl.when((k & 1) == 1)
        def _():
            chunk_ops(k, set1, set0)

        return carry

    lax.fori_loop(0, nk, body, jnp.int32(0))

    last = (nk - 1) & 1

    @pl.when(last == 0)
    def _():
        scatter_drain(set0)

    @pl.when(last == 1)
    def _():
        scatter_drain(set1)

    pltpu.sync_copy(cbuf.at[pl.ds(32, 16)], epart.at[pl.ds(wid * 16, 16)])

    plsc.subcore_barrier()
    pltpu.sync_copy(facc.at[pl.ds(s * 3 * SLC, 3 * SLC)],
                    fpart.at[pl.ds(c * 3 * NP + s * 3 * SLC, 3 * SLC)])


def _tc_combine(a_ref, b_ref, e_ref, o_ref, eo_ref):
    o_ref[...] = a_ref[...] + b_ref[...]
    eo_ref[...] = jnp.sum(e_ref[...]).reshape(1, 1)


def kernel(pos, edge_index, epsilon, sigma):
    pos = pos.astype(_f32)
    px = jnp.pad(pos[:, 0], (0, NP - N_NODES))
    py = jnp.pad(pos[:, 1], (0, NP - N_NODES))
    pz = jnp.pad(pos[:, 2], (0, NP - N_NODES))
    eif = edge_index.reshape(-1)
    consts = jnp.concatenate([jnp.full((16,), 4.0 * epsilon, _f32),
                              jnp.full((16,), sigma, _f32)])
    zeros = jnp.zeros((3 * SLC,), _f32)

    mesh = plsc.VectorSubcoreMesh(core_axis_name="c", subcore_axis_name="s")

    fpart, epart = pl.kernel(
        _edge_pass,
        out_type=[
            jax.ShapeDtypeStruct((2 * 3 * NP,), _f32),
            jax.ShapeDtypeStruct((NW * 16,), _f32),
        ],
        mesh=mesh,
        scratch_types=(
            [pltpu.VMEM((2 * CH,), _i32)] * 4
            + [pltpu.VMEM((2 * CH,), _f32)] * 6
            + [pltpu.VMEM((2 * CH,), _i32)] * 4
            + [pltpu.VMEM((2 * CH,), _f32)] * 6
            + [
                pltpu.VMEM((48,), _f32),
                pltpu.VMEM_SHARED((3 * NP,), _f32),
                pltpu.VMEM_SHARED((NP,), _f32),
                pltpu.VMEM_SHARED((NP,), _f32),
                pltpu.VMEM_SHARED((NP,), _f32),
            ]
            + [pltpu.SemaphoreType.DMA] * 6
        ),
    )(eif, px, py, pz, consts, zeros)

    rows = 3 * NP // 128
    a2 = fpart[:3 * NP].reshape(rows, 128)
    b2 = fpart[3 * NP:].reshape(rows, 128)
    e2 = epart.reshape(4, 128)
    force, etot = pl.pallas_call(
        _tc_combine,
        out_shape=[
            jax.ShapeDtypeStruct((rows, 128), _f32),
            jax.ShapeDtypeStruct((1, 1), _f32),
        ],
    )(a2, b2, e2)

    force = force.reshape(3 * NP)[:N_NODES * 3].reshape(N_NODES, 3)
    return etot[0, 0], force


# speedup vs baseline: 1.1976x; 1.0002x over previous
import jax
import jax.numpy as jnp
from jax import lax
from jax.experimental import pallas as pl
from jax.experimental.pallas import tpu as pltpu
from jax.experimental.pallas import tpu_sc as plsc

N_NODES = 100000
N_EDGES = 6400000

NW = 32
CH = 2048
NCH = N_EDGES // CH
NP = 100352
SLC = NP // 16
R2 = NP // NW

_f32 = jnp.float32
_i32 = jnp.int32


def _rsqrt(t):
    bits = lax.bitcast_convert_type(t, _i32)
    y = lax.bitcast_convert_type(jnp.int32(0x5F3759DF) - (bits >> 1), _f32)
    for _ in range(2):
        y = y * (1.5 - 0.5 * t * y * y)
    return y


def _edge_pass(eif, px, py, pz, consts, zeros, fpart, epart,
               cat0, cx0, cy0, cz0, bx0, by0, bz0, gx0, gy0, gz0,
               cat1, cx1, cy1, cz1, bx1, by1, bz1, gx1, gy1, gz1,
               cbuf, facc, spx, spy, spz,
               sg0, sg1, ss0, ss1, sj0, sj1):
    c = lax.axis_index("c")
    s = lax.axis_index("s")
    wid = s * 2 + c

    set0 = (cat0, cx0, cy0, cz0, bx0, by0, bz0, gx0, gy0, gz0, sg0, ss0, sj0)
    set1 = (cat1, cx1, cy1, cz1, bx1, by1, bz1, gx1, gy1, gz1, sg1, ss1, sj1)

    sl = pl.ds(s * SLC, SLC)
    pltpu.sync_copy(zeros.at[pl.ds(0, 3 * SLC)],
                    facc.at[pl.ds(s * 3 * SLC, 3 * SLC)])
    pltpu.sync_copy(px.at[sl], spx.at[sl])
    pltpu.sync_copy(py.at[sl], spy.at[sl])
    pltpu.sync_copy(pz.at[sl], spz.at[sl])
    pltpu.sync_copy(consts, cbuf.at[pl.ds(0, 32)])
    cbuf[pl.ds(32, 16)] = jnp.zeros((16,), _f32)
    plsc.subcore_barrier()

    eps4 = cbuf[pl.ds(0, 16)]
    sig = cbuf[pl.ds(16, 16)]

    nk = (NCH - wid + NW - 1) // NW

    def gather_descs(S):
        cat, bx, by, bz = S[0], S[4], S[5], S[6]
        return [pltpu.make_async_copy(spx.at[cat], bx, S[10]),
                pltpu.make_async_copy(spy.at[cat], by, S[10]),
                pltpu.make_async_copy(spz.at[cat], bz, S[10])]

    def scatter_drain(S):
        cx, cy, cz, gx, gy, gz = S[1], S[2], S[3], S[7], S[8], S[9]
        pltpu.make_async_copy(gx, facc.at[cx], S[11]).wait()
        pltpu.make_async_copy(gy, facc.at[cy], S[11]).wait()
        pltpu.make_async_copy(gz, facc.at[cz], S[11]).wait()

    def scatter_issue(S):
        cx, cy, cz, gx, gy, gz = S[1], S[2], S[3], S[7], S[8], S[9]
        pltpu.async_copy(gx, facc.at[cx], S[11], add=True)
        pltpu.async_copy(gy, facc.at[cy], S[11], add=True)
        pltpu.async_copy(gz, facc.at[cz], S[11], add=True)

    def idx_issue(S, j):
        cat = S[0]
        pltpu.async_copy(eif.at[pl.ds(j * CH, CH)], cat.at[pl.ds(0, CH)], S[12])
        pltpu.async_copy(eif.at[pl.ds(N_EDGES + j * CH, CH)],
                         cat.at[pl.ds(CH, CH)], S[12])

    def idx_drain(S):
        cat = S[0]
        pltpu.make_async_copy(eif.at[pl.ds(0, CH)], cat.at[pl.ds(0, CH)], S[12]).wait()
        pltpu.make_async_copy(eif.at[pl.ds(0, CH)], cat.at[pl.ds(CH, CH)], S[12]).wait()

    def compute(S):
        cat, cx, cy, cz = S[0], S[1], S[2], S[3]
        bx, by, bz, gx, gy, gz = S[4], S[5], S[6], S[7], S[8], S[9]

        def blk(b, acc):
            cs = pl.ds(b * 16, 16)
            cd = pl.ds(CH + b * 16, 16)
            vs3 = cat[cs] * 3
            cx[cs] = vs3
            cy[cs] = vs3 + 1
            cz[cs] = vs3 + 2
            vd3 = cat[cd] * 3
            cx[cd] = vd3
            cy[cd] = vd3 + 1
            cz[cd] = vd3 + 2
            dx = bx[cs] - bx[cd]
            dy = by[cs] - by[cd]
            dz = bz[cs] - bz[cd]
            t = dx * dx + dy * dy + dz * dz + 1e-12
            rin = _rsqrt(t)
            rr = t * rin
            qi = 1.0 / (rr + 1.0)
            inv = sig * qi
            i2 = inv * inv
            i6 = i2 * i2 * i2
            i12 = i6 * i6
            acc = acc + eps4 * (i12 - i6)
            dvdr = eps4 * (6.0 * i6 - 12.0 * i12) * qi
            cf = -0.5 * dvdr * rin
            hx = cf * dx
            hy = cf * dy
            hz = cf * dz
            gx[cs] = hx
            gy[cs] = hy
            gz[cs] = hz
            gx[cd] = -hx
            gy[cd] = -hy
            gz[cd] = -hz
            return acc

        eacc = lax.fori_loop(0, CH // 16, blk, cbuf[pl.ds(32, 16)])
        cbuf[pl.ds(32, 16)] = eacc

    def chunk_ops(k, S, T):
        @pl.when(k + 1 < nk)
        def _():
            idx_issue(T, wid + (k + 1) * NW)
            idx_drain(T)
            for d in gather_descs(T):
                d.start()

        for d in gather_descs(S):
            d.wait()

        compute(S)

        @pl.when(k >= 1)
        def _():
            scatter_drain(T)

        scatter_issue(S)

    pltpu.sync_copy(eif.at[pl.ds(wid * CH, CH)], cat0.at[pl.ds(0, CH)])
    pltpu.sync_copy(eif.at[pl.ds(N_EDGES + wid * CH, CH)],
                    cat0.at[pl.ds(CH, CH)])
    for d in gather_descs(set0):
        d.start()

    def body(k, carry):
        @pl.when((k & 1) == 0)
        def _():
            chunk_ops(k, set0, set1)

        @pl.when((k & 1) == 1)
        def _():
            chunk_ops(k, set1, set0)

        return carry

    lax.fori_loop(0, nk, body, jnp.int32(0))

    last = (nk - 1) & 1

    @pl.when(last == 0)
    def _():
        scatter_drain(set0)

    @pl.when(last == 1)
    def _():
        scatter_drain(set1)

    pltpu.sync_copy(cbuf.at[pl.ds(32, 16)], epart.at[pl.ds(wid * 16, 16)])

    plsc.subcore_barrier()
    pltpu.sync_copy(facc.at[pl.ds(s * 3 * SLC, 3 * SLC)],
                    fpart.at[pl.ds(c * 3 * NP + s * 3 * SLC, 3 * SLC)])


def _tc_combine(a_ref, b_ref, e_ref, o_ref, eo_ref):
    o_ref[...] = a_ref[...] + b_ref[...]
    eo_ref[...] = jnp.sum(e_ref[...]).reshape(1, 1)


def kernel(pos, edge_index, epsilon, sigma):
    pos = pos.astype(_f32)
    px = jnp.pad(pos[:, 0], (0, NP - N_NODES))
    py = jnp.pad(pos[:, 1], (0, NP - N_NODES))
    pz = jnp.pad(pos[:, 2], (0, NP - N_NODES))
    eif = edge_index.reshape(-1)
    consts = jnp.concatenate([jnp.full((16,), 4.0 * epsilon, _f32),
                              jnp.full((16,), sigma, _f32)])
    zeros = jnp.zeros((3 * SLC,), _f32)

    mesh = plsc.VectorSubcoreMesh(core_axis_name="c", subcore_axis_name="s")

    fpart, epart = pl.kernel(
        _edge_pass,
        out_type=[
            jax.ShapeDtypeStruct((2 * 3 * NP,), _f32),
            jax.ShapeDtypeStruct((NW * 16,), _f32),
        ],
        mesh=mesh,
        scratch_types=(
            [pltpu.VMEM((2 * CH,), _i32)] * 4
            + [pltpu.VMEM((2 * CH,), _f32)] * 6
            + [pltpu.VMEM((2 * CH,), _i32)] * 4
            + [pltpu.VMEM((2 * CH,), _f32)] * 6
            + [
                pltpu.VMEM((48,), _f32),
                pltpu.VMEM_SHARED((3 * NP,), _f32),
                pltpu.VMEM_SHARED((NP,), _f32),
                pltpu.VMEM_SHARED((NP,), _f32),
                pltpu.VMEM_SHARED((NP,), _f32),
            ]
            + [pltpu.SemaphoreType.DMA] * 6
        ),
    )(eif, px, py, pz, consts, zeros)

    rows = 3 * NP // 128
    a2 = fpart[:3 * NP].reshape(rows, 128)
    b2 = fpart[3 * NP:].reshape(rows, 128)
    e2 = epart.reshape(4, 128)
    force, etot = pl.pallas_call(
        _tc_combine,
        out_shape=[
            jax.ShapeDtypeStruct((rows, 128), _f32),
            jax.ShapeDtypeStruct((1, 1), _f32),
        ],
    )(a2, b2, e2)

    force = force.reshape(3 * NP)[:N_NODES * 3].reshape(N_NODES, 3)
    return etot[0, 0], force
